# baseline (speedup 1.0000x reference)
import functools

import jax
import jax.numpy as jnp
from jax import lax
from jax.experimental import pallas as pl
from jax.experimental.pallas import tpu as pltpu
from jax.experimental.pallas import tpu_sc as plsc

_IDX_CHUNK = 128


@functools.lru_cache(maxsize=None)
def _build(B, V, D):
    info = plsc.get_sparse_core_info()
    NC, NS, L = info.num_cores, info.num_subcores, info.num_lanes
    NW = NC * NS
    assert B % NW == 0
    b_per_w = B // NW
    n_chunks = b_per_w // _IDX_CHUNK
    assert n_chunks * _IDX_CHUNK == b_per_w
    mesh = plsc.VectorSubcoreMesh(core_axis_name="c", subcore_axis_name="s")

    @functools.partial(
        pl.kernel,
        mesh=mesh,
        out_type=jax.ShapeDtypeStruct((B, D), jnp.float32),
        compiler_params=pltpu.CompilerParams(use_tc_tiling_on_sc=False),
        scratch_types=[
            pltpu.VMEM((n_chunks, _IDX_CHUNK), jnp.int32),
            pltpu.VMEM((n_chunks, _IDX_CHUNK), jnp.int32),
            pltpu.VMEM((b_per_w, D), jnp.float32),
            pltpu.VMEM((b_per_w, D), jnp.float32),
            pltpu.SemaphoreType.DMA,
            pltpu.SemaphoreType.DMA,
        ],
    )
    def gmf(uid_hbm, iid_hbm, ut_hbm, it_hbm, out_hbm,
            uidx_v, iidx_v, urows_v, irows_v, sem_u, sem_i):
        wid = lax.axis_index("s") * NC + lax.axis_index("c")
        base = wid * b_per_w
        pltpu.sync_copy(uid_hbm.at[wid], uidx_v)
        pltpu.sync_copy(iid_hbm.at[wid], iidx_v)
        copies = []
        for j in range(n_chunks):
            rows = pl.ds(j * _IDX_CHUNK, _IDX_CHUNK)
            copies.append(
                pltpu.async_copy(ut_hbm.at[uidx_v.at[j]], urows_v.at[rows], sem_u))
            copies.append(
                pltpu.async_copy(it_hbm.at[iidx_v.at[j]], irows_v.at[rows], sem_i))
        for c in copies:
            c.wait()

        def mul_row(i, carry):
            for h in range(D // L):
                s = pl.ds(h * L, L)
                urows_v[i, s] = urows_v[i, s] * irows_v[i, s]
            return carry

        lax.fori_loop(0, b_per_w, mul_row, 0)
        pltpu.sync_copy(urows_v, out_hbm.at[pl.ds(base, b_per_w)])

    def run(user_ids, item_ids, user_table, item_table):
        uid = user_ids.astype(jnp.int32).reshape(NW, n_chunks, _IDX_CHUNK)
        iid = item_ids.astype(jnp.int32).reshape(NW, n_chunks, _IDX_CHUNK)
        return gmf(uid, iid, user_table, item_table)

    return run


@jax.jit
def kernel(user_ids, item_ids, user_table, item_table):
    (B,) = user_ids.shape
    V, D = user_table.shape
    return _build(B, V, D)(user_ids, item_ids, user_table, item_table)
